# baseline (speedup 1.0000x reference)
import jax
import jax.numpy as jnp
from jax.experimental import pallas as pl

_N = 4096
_D = 64
_BLK = 256
_NBLK = _N // _BLK

_ALPHA = 1.0


def _rowsum(x, ones):
    return jax.lax.dot_general(
        x, ones, (((1,), (0,)), ((), ())), preferred_element_type=jnp.float32
    )


def _order_stats_bits(bits, ones, ks):
    blk = bits.shape[0]
    los = [jnp.zeros((blk, 1), jnp.int32) for _ in ks]
    his = [jnp.full((blk, 1), 0x7F800000, jnp.int32) for _ in ks]

    def it(_, carry):
        los, his = carry
        nlos, nhis = [], []
        for k, lo, hi in zip(ks, los, his):
            mid = lo + (hi - lo) // 2
            ind = jnp.where(bits <= mid, 1.0, 0.0)
            cnt = _rowsum(ind, ones)
            ge = cnt >= k
            nlos.append(jnp.where(ge, lo, mid + 1))
            nhis.append(jnp.where(ge, mid, hi))
        return nlos, nhis

    los, his = jax.lax.fori_loop(0, 31, it, (los, his))
    return los


def _body(x_ref, xb_ref, reg_ref, l32_ref, l512_ref):
    i = pl.program_id(0)
    x = x_ref[...]
    xb = xb_ref[...]
    x2 = jnp.sum(x * x, axis=1)
    xb2 = jnp.sum(xb * xb, axis=1)
    g = jax.lax.dot_general(
        xb, x, (((1,), (1,)), ((), ())), preferred_element_type=jnp.float32
    )
    d2 = xb2[:, None] + x2[None, :] - 2.0 * g
    d2c = jnp.maximum(d2, 1e-12)
    bits = jax.lax.bitcast_convert_type(d2c, jnp.int32)
    ones = jnp.ones((_N, 1), jnp.float32)

    t21b, t33b, t513b = _order_stats_bits(bits, ones, (21, 33, 513))
    t21 = jax.lax.bitcast_convert_type(t21b, jnp.float32)
    t33 = jax.lax.bitcast_convert_type(t33b, jnp.float32)
    t513 = jax.lax.bitcast_convert_type(t513b, jnp.float32)

    s = jnp.sqrt(d2c)
    lg = 0.5 * jnp.log(d2c)
    m21 = d2c < t21
    m33 = d2c < t33
    m513 = d2c < t513
    i21 = jnp.where(m21, 1.0, 0.0)
    i33 = jnp.where(m33, 1.0, 0.0)
    i513 = jnp.where(m513, 1.0, 0.0)
    c21 = _rowsum(i21, ones)
    c33 = _rowsum(i33, ones)
    c513 = _rowsum(i513, ones)
    s1 = _rowsum(i21 * s, ones)
    s2 = _rowsum(i33 * lg, ones)
    s3 = _rowsum(i513 * lg, ones)
    a0 = jnp.min(s, axis=1, keepdims=True)
    log_a0 = jnp.log(a0)

    sq21 = jnp.sqrt(t21)
    lg33 = 0.5 * jnp.log(t33)
    lg513 = 0.5 * jnp.log(t513)

    s20 = s1 + (20.0 - c21) * sq21
    m = (s20 - a0) / 19.0
    lid_mom = m / (sq21 - m)
    reg_row = -jnp.abs(jnp.log(lid_mom))

    l32sum = s2 + (32.0 - c33) * lg33
    lids32 = -32.0 / (l32sum - log_a0 - 31.0 * lg33)
    l512sum = s3 + (512.0 - c513) * lg513
    lids512 = -512.0 / (l512sum - log_a0 - 511.0 * lg513)

    l32_ref[...] = lids32
    l512_ref[...] = lids512

    @pl.when(i == 0)
    def _():
        reg_ref[...] = jnp.zeros_like(reg_ref)

    reg_ref[...] += jnp.sum(reg_row, axis=(0, 1), keepdims=True)


def kernel(features):
    reg_sum, l32, l512 = pl.pallas_call(
        _body,
        grid=(_NBLK,),
        in_specs=[
            pl.BlockSpec((_N, _D), lambda i: (0, 0)),
            pl.BlockSpec((_BLK, _D), lambda i: (i, 0)),
        ],
        out_specs=[
            pl.BlockSpec((1, 1), lambda i: (0, 0)),
            pl.BlockSpec((_BLK, 1), lambda i: (i, 0)),
            pl.BlockSpec((_BLK, 1), lambda i: (i, 0)),
        ],
        out_shape=[
            jax.ShapeDtypeStruct((1, 1), jnp.float32),
            jax.ShapeDtypeStruct((_N, 1), jnp.float32),
            jax.ShapeDtypeStruct((_N, 1), jnp.float32),
        ],
    )(features, features)
    reg_loss = _ALPHA * reg_sum[0, 0] / _N
    return (reg_loss, l32[:, 0], l512[:, 0])


# speedup vs baseline: 1.1504x; 1.1504x over previous
import jax
import jax.numpy as jnp
from jax.experimental import pallas as pl

_N = 4096
_D = 64
_BLK = 256
_NBLK = _N // _BLK

_ALPHA = 1.0


def _order_stats_bits(bits, ks):
    blk = bits.shape[0]
    los = [jnp.zeros((blk, 1), jnp.int32) for _ in ks]
    his = [jnp.full((blk, 1), 0x7F800000, jnp.int32) for _ in ks]

    def it(_, carry):
        los, his = carry
        nlos, nhis = [], []
        for k, lo, hi in zip(ks, los, his):
            mid = lo + (hi - lo) // 2
            cnt = jnp.sum((bits <= mid).astype(jnp.int32), axis=1, keepdims=True)
            ge = cnt >= k
            nlos.append(jnp.where(ge, lo, mid + 1))
            nhis.append(jnp.where(ge, mid, hi))
        return nlos, nhis

    los, his = jax.lax.fori_loop(0, 31, it, (los, his))
    return los


def _body(x_ref, xb_ref, reg_ref, l32_ref, l512_ref):
    i = pl.program_id(0)
    x = x_ref[...]
    xb = xb_ref[...]
    x2 = jnp.sum(x * x, axis=1)
    xb2 = jnp.sum(xb * xb, axis=1)
    g = jax.lax.dot_general(
        xb, x, (((1,), (1,)), ((), ())), preferred_element_type=jnp.float32
    )
    d2 = xb2[:, None] + x2[None, :] - 2.0 * g
    d2c = jnp.maximum(d2, 1e-12)
    bits = jax.lax.bitcast_convert_type(d2c, jnp.int32)

    t21b, t33b, t513b = _order_stats_bits(bits, (21, 33, 513))
    t21 = jax.lax.bitcast_convert_type(t21b, jnp.float32)
    t33 = jax.lax.bitcast_convert_type(t33b, jnp.float32)
    t513 = jax.lax.bitcast_convert_type(t513b, jnp.float32)

    s = jnp.sqrt(d2c)
    lg = 0.5 * jnp.log(d2c)
    m21 = d2c < t21
    m33 = d2c < t33
    m513 = d2c < t513
    c21 = jnp.sum(m21.astype(jnp.float32), axis=1, keepdims=True)
    c33 = jnp.sum(m33.astype(jnp.float32), axis=1, keepdims=True)
    c513 = jnp.sum(m513.astype(jnp.float32), axis=1, keepdims=True)
    s1 = jnp.sum(jnp.where(m21, s, 0.0), axis=1, keepdims=True)
    s2 = jnp.sum(jnp.where(m33, lg, 0.0), axis=1, keepdims=True)
    s3 = jnp.sum(jnp.where(m513, lg, 0.0), axis=1, keepdims=True)
    a0 = jnp.min(s, axis=1, keepdims=True)
    log_a0 = jnp.log(a0)

    sq21 = jnp.sqrt(t21)
    lg33 = 0.5 * jnp.log(t33)
    lg513 = 0.5 * jnp.log(t513)

    s20 = s1 + (20.0 - c21) * sq21
    m = (s20 - a0) / 19.0
    lid_mom = m / (sq21 - m)
    reg_row = -jnp.abs(jnp.log(lid_mom))

    l32sum = s2 + (32.0 - c33) * lg33
    lids32 = -32.0 / (l32sum - log_a0 - 31.0 * lg33)
    l512sum = s3 + (512.0 - c513) * lg513
    lids512 = -512.0 / (l512sum - log_a0 - 511.0 * lg513)

    l32_ref[...] = lids32
    l512_ref[...] = lids512

    @pl.when(i == 0)
    def _():
        reg_ref[...] = jnp.zeros_like(reg_ref)

    reg_ref[...] += jnp.sum(reg_row, axis=(0, 1), keepdims=True)


def kernel(features):
    reg_sum, l32, l512 = pl.pallas_call(
        _body,
        grid=(_NBLK,),
        in_specs=[
            pl.BlockSpec((_N, _D), lambda i: (0, 0)),
            pl.BlockSpec((_BLK, _D), lambda i: (i, 0)),
        ],
        out_specs=[
            pl.BlockSpec((1, 1), lambda i: (0, 0)),
            pl.BlockSpec((_BLK, 1), lambda i: (i, 0)),
            pl.BlockSpec((_BLK, 1), lambda i: (i, 0)),
        ],
        out_shape=[
            jax.ShapeDtypeStruct((1, 1), jnp.float32),
            jax.ShapeDtypeStruct((_N, 1), jnp.float32),
            jax.ShapeDtypeStruct((_N, 1), jnp.float32),
        ],
    )(features, features)
    reg_loss = _ALPHA * reg_sum[0, 0] / _N
    return (reg_loss, l32[:, 0], l512[:, 0])
